# baseline (speedup 1.0000x reference)
import functools

import jax
import jax.numpy as jnp

from jax import lax
from jax.experimental import pallas as pl
from jax.experimental.pallas import tpu as pltpu
from jax.experimental.pallas import tpu_sc as plsc

_N = 10000
_E = 320000
_D = 128
_EMB = 16
_CONT = 12

_BN = 2000
_NB = _N // _BN

_WORKERS = 32
_CHUNK = 125
_NCHT = _E // _CHUNK
_NCH = _NCHT // _WORKERS
_SCH = 16
_NSCH = _NCH // _SCH
_TILES = 16
_NPAD = 10240
_RPT = _NPAD // _TILES


def _input_body(idx_ref, cont_ref, me_ref, we_ref, te_ref,
                w1m_ref, w1w_ref, w1t_ref, w1c_ref, b1_ref, w2t_ref, b2_ref,
                out_ref):
    f32 = jnp.float32
    m = idx_ref[:, 0:1]
    w = idx_ref[:, 1:2]
    t = idx_ref[:, 2:3]
    ohm = (m == lax.broadcasted_iota(jnp.int32, (_BN, 8), 1)).astype(f32)
    ohw = (w == lax.broadcasted_iota(jnp.int32, (_BN, 7), 1)).astype(f32)
    oht = (t == lax.broadcasted_iota(jnp.int32, (_BN, 512), 1)).astype(f32)
    tm = jnp.dot(me_ref[...], w1m_ref[...], preferred_element_type=f32)
    tw = jnp.dot(we_ref[...], w1w_ref[...], preferred_element_type=f32)
    tt = jnp.dot(te_ref[...], w1t_ref[...], preferred_element_type=f32)
    z1 = (jnp.dot(ohm, tm, preferred_element_type=f32)
          + jnp.dot(ohw, tw, preferred_element_type=f32)
          + jnp.dot(oht, tt, preferred_element_type=f32)
          + jnp.dot(cont_ref[...], w1c_ref[...], preferred_element_type=f32)
          + b1_ref[...])
    h1 = jnp.maximum(z1, 0.0)
    z2 = jnp.dot(h1, w2t_ref[...], preferred_element_type=f32) + b2_ref[...]
    out_ref[...] = jnp.maximum(z2, 0.0)


def _input_stage(idx3, cont, me, we, te, w1m, w1w, w1t, w1c, b1, w2t, b2):
    full = lambda shape: pl.BlockSpec(shape, lambda i: (0, 0))
    return pl.pallas_call(
        _input_body,
        grid=(_NB,),
        in_specs=[
            pl.BlockSpec((_BN, 3), lambda i: (i, 0)),
            pl.BlockSpec((_BN, _CONT), lambda i: (i, 0)),
            full((8, _EMB)), full((7, _EMB)), full((512, _EMB)),
            full((_EMB, _D)), full((_EMB, _D)), full((_EMB, _D)),
            full((_CONT, _D)), full((1, _D)), full((_D, _D)), full((1, _D)),
        ],
        out_specs=pl.BlockSpec((_BN, _D), lambda i: (i, 0)),
        out_shape=jax.ShapeDtypeStruct((_N, _D), jnp.float32),
    )(idx3, cont, me, we, te, w1m, w1w, w1t, w1c, b1, w2t, b2)


def _segsum_body(h_hbm, col_hbm, row_hbm, val_hbm, zero_hbm, dumm_hbm, out_hbm,
                 col_sc, row_sc, valv, buf0, buf1, acc,
                 gsem0, gsem1, ssem0, ssem1):
    cid = lax.axis_index("c")
    sid = lax.axis_index("s")
    wid = cid * _TILES + sid

    pltpu.sync_copy(zero_hbm, acc.at[pl.ds(sid * _RPT, _RPT)])
    plsc.subcore_barrier()

    _T = _SCH // 2

    def drain(buf, sem):
        pltpu.make_async_copy(dumm_hbm, buf, sem).wait()

    def scale(buf, j):
        def emit_edge(vg, e, k):
            wv = jnp.full((16,), vg[k], jnp.float32)
            for c in range(_D // 16):
                sl = pl.ds(c * 16, 16)
                buf[e, sl] = buf[e, sl] * wv

        def group_body(g, carry):
            vg = valv[pl.ds(j * 128 + g * 16, 16)]
            for k in range(16):
                emit_edge(vg, g * 16 + k, k)
            return carry
        lax.fori_loop(0, 7, group_body, 0)
        vg = valv[pl.ds(j * 128 + 112, 16)]
        for k in range(_CHUNK - 112):
            emit_edge(vg, 112 + k, k)

    def sch_body(s, carry):
        cbase = wid * _NCH + s * _SCH
        pltpu.sync_copy(col_hbm.at[pl.ds(cbase, _SCH)], col_sc)
        pltpu.sync_copy(row_hbm.at[pl.ds(cbase, _SCH)], row_sc)
        pltpu.sync_copy(val_hbm.at[pl.ds(cbase * 128, _SCH * 128)], valv)
        pltpu.async_copy(h_hbm.at[col_sc.at[0]],
                         buf0.at[pl.ds(0, _CHUNK)], gsem0)

        def pair_body(t, carry2):
            j0 = 2 * t
            j1 = 2 * t + 1

            @pl.when(t > 0)
            def _():
                drain(buf1, ssem1)
            pltpu.async_copy(h_hbm.at[col_sc.at[j0 + 1]],
                             buf1.at[pl.ds(0, _CHUNK)], gsem1)
            drain(buf0, gsem0)
            scale(buf0, j0)
            pltpu.async_copy(buf0.at[pl.ds(0, _CHUNK)],
                             acc.at[row_sc.at[j0]], ssem0, add=True)

            @pl.when(t < _T - 1)
            def _():
                drain(buf0, ssem0)
                pltpu.async_copy(h_hbm.at[col_sc.at[j1 + 1]],
                                 buf0.at[pl.ds(0, _CHUNK)], gsem0)
            drain(buf1, gsem1)
            scale(buf1, j1)
            pltpu.async_copy(buf1.at[pl.ds(0, _CHUNK)],
                             acc.at[row_sc.at[j1]], ssem1, add=True)
            return carry2
        lax.fori_loop(0, _T, pair_body, 0)
        drain(buf1, ssem1)
        drain(buf0, ssem0)
        return carry
    lax.fori_loop(0, _NSCH, sch_body, 0)

    plsc.subcore_barrier()
    pltpu.sync_copy(acc.at[pl.ds(sid * _RPT, _RPT)],
                    out_hbm.at[cid, pl.ds(sid * _RPT, _RPT)])


def _make_segsum():
    mesh = plsc.VectorSubcoreMesh(core_axis_name="c", subcore_axis_name="s")
    return functools.partial(
        pl.kernel, _segsum_body, mesh=mesh,
        out_type=jax.ShapeDtypeStruct((2, _NPAD, _D), jnp.float32),
        scratch_types=[
            pltpu.VMEM((_SCH, _CHUNK), jnp.int32),
            pltpu.VMEM((_SCH, _CHUNK), jnp.int32),
            pltpu.VMEM((_SCH * 128,), jnp.float32),
            pltpu.VMEM((_CHUNK, _D), jnp.float32),
            pltpu.VMEM((_CHUNK, _D), jnp.float32),
            pltpu.VMEM_SHARED((_NPAD, _D), jnp.float32),
            pltpu.SemaphoreType.DMA,
            pltpu.SemaphoreType.DMA,
            pltpu.SemaphoreType.DMA,
            pltpu.SemaphoreType.DMA,
        ],
    )()


def _selfz_body(h_ref, swt_ref, sb_ref, out_ref):
    f32 = jnp.float32
    out_ref[...] = (jnp.dot(h_ref[...], swt_ref[...],
                            preferred_element_type=f32) + sb_ref[...])


def _selfz(h, swt, sb):
    full = lambda shape: pl.BlockSpec(shape, lambda i: (0, 0))
    return pl.pallas_call(
        _selfz_body, grid=(_NB,),
        in_specs=[pl.BlockSpec((_BN, _D), lambda i: (i, 0)),
                  full((_D, _D)), full((1, _D))],
        out_specs=pl.BlockSpec((_BN, _D), lambda i: (i, 0)),
        out_shape=jax.ShapeDtypeStruct((_N, _D), jnp.float32),
    )(h, swt, sb)


def _layer_body(h_ref, p_ref, sz_ref, nwt_ref, nb_ref, out_ref):
    f32 = jnp.float32
    h = h_ref[...]
    neigh = p_ref[0] + p_ref[1]
    z = (sz_ref[...]
         + jnp.dot(neigh, nwt_ref[...], preferred_element_type=f32)
         + nb_ref[...])
    out_ref[...] = h + jnp.maximum(z, 0.0)


def _layer_final_body(h_ref, p_ref, sz_ref, nwt_ref, nb_ref,
                      out_ref, gsum_ref, gmax_ref):
    f32 = jnp.float32
    i = pl.program_id(0)
    h = h_ref[...]
    neigh = p_ref[0] + p_ref[1]
    z = (sz_ref[...]
         + jnp.dot(neigh, nwt_ref[...], preferred_element_type=f32)
         + nb_ref[...])
    hn = h + jnp.maximum(z, 0.0)
    out_ref[...] = hn
    bs = jnp.sum(hn, axis=0, keepdims=True)
    bm = jnp.max(hn, axis=0, keepdims=True)

    @pl.when(i == 0)
    def _():
        gsum_ref[...] = bs
        gmax_ref[...] = bm

    @pl.when(i > 0)
    def _():
        gsum_ref[...] = gsum_ref[...] + bs
        gmax_ref[...] = jnp.maximum(gmax_ref[...], bm)

    @pl.when(i == _NB - 1)
    def _():
        gsum_ref[...] = gsum_ref[...] * (1.0 / _N)


def _layer(h, p, sz, nwt, nb, final):
    full = lambda shape: pl.BlockSpec(shape, lambda i: (0, 0))
    in_specs = [
        pl.BlockSpec((_BN, _D), lambda i: (i, 0)),
        pl.BlockSpec((2, _BN, _D), lambda i: (0, i, 0)),
        pl.BlockSpec((_BN, _D), lambda i: (i, 0)),
        full((_D, _D)), full((1, _D)),
    ]
    if not final:
        return pl.pallas_call(
            _layer_body, grid=(_NB,), in_specs=in_specs,
            out_specs=pl.BlockSpec((_BN, _D), lambda i: (i, 0)),
            out_shape=jax.ShapeDtypeStruct((_N, _D), jnp.float32),
        )(h, p, sz, nwt, nb)
    return pl.pallas_call(
        _layer_final_body, grid=(_NB,), in_specs=in_specs,
        out_specs=[
            pl.BlockSpec((_BN, _D), lambda i: (i, 0)),
            pl.BlockSpec((1, _D), lambda i: (0, 0)),
            pl.BlockSpec((1, _D), lambda i: (0, 0)),
        ],
        out_shape=[
            jax.ShapeDtypeStruct((_N, _D), jnp.float32),
            jax.ShapeDtypeStruct((1, _D), jnp.float32),
            jax.ShapeDtypeStruct((1, _D), jnp.float32),
        ],
    )(h, p, sz, nwt, nb)


_segsum = None


def kernel(machine_idx, weekday_idx, type_idx, cont_feat, a_indices, a_values,
           machine_emb, weekday_emb, type_emb, W_in1, b_in1, W_in2, b_in2,
           self_W, self_b, neigh_W, neigh_b):
    global _segsum
    if _segsum is None:
        _segsum = _make_segsum()

    row = a_indices[0]
    col = a_indices[1]
    idx3 = jnp.stack([machine_idx.astype(jnp.int32),
                      weekday_idx.astype(jnp.int32),
                      type_idx.astype(jnp.int32)], axis=1)

    w1m = W_in1[:, 0:_EMB].T
    w1w = W_in1[:, _EMB:2 * _EMB].T
    w1t = W_in1[:, 2 * _EMB:3 * _EMB].T
    w1c = W_in1[:, 3 * _EMB:].T
    b1 = b_in1.reshape(1, _D)
    w2t = W_in2.T
    b2 = b_in2.reshape(1, _D)

    h = _input_stage(idx3, cont_feat, machine_emb, weekday_emb,
                     type_emb, w1m, w1w, w1t, w1c, b1, w2t, b2)

    zeros = jnp.zeros((_RPT, _D), jnp.float32)
    dummy = jnp.zeros((_CHUNK, _D), jnp.float32)
    col2 = col.reshape(_NCHT, _CHUNK)
    row2 = row.reshape(_NCHT, _CHUNK)
    vals2 = jnp.pad(a_values.reshape(_NCHT, _CHUNK),
                    ((0, 0), (0, 128 - _CHUNK))).reshape(_NCHT * 128)
    for l in range(2):
        sz = _selfz(h, self_W[l].T, self_b[l].reshape(1, _D))
        p = _segsum(h, col2, row2, vals2, zeros, dummy)
        nwt = neigh_W[l].T
        nb = neigh_b[l].reshape(1, _D)
        if l == 0:
            h = _layer(h, p, sz, nwt, nb, final=False)
        else:
            h, gsum, gmax = _layer(h, p, sz, nwt, nb, final=True)

    g = jnp.concatenate([gsum[0], gmax[0]], axis=0)
    return (g, h)


# speedup vs baseline: 1.0016x; 1.0016x over previous
import functools

import jax
import jax.numpy as jnp

from jax import lax
from jax.experimental import pallas as pl
from jax.experimental.pallas import tpu as pltpu
from jax.experimental.pallas import tpu_sc as plsc

_N = 10000
_E = 320000
_D = 128
_EMB = 16
_CONT = 12

_BN = 2000
_NB = _N // _BN

_WORKERS = 32
_CHUNK = 125
_NCHT = _E // _CHUNK
_NCH = _NCHT // _WORKERS
_SCH = 16
_NSCH = _NCH // _SCH
_TILES = 16
_NPAD = 10240
_RPT = _NPAD // _TILES


def _input_body(idx_ref, cont_ref, me_ref, we_ref, te_ref,
                w1m_ref, w1w_ref, w1t_ref, w1c_ref, b1_ref, w2t_ref, b2_ref,
                out_ref):
    f32 = jnp.float32
    m = idx_ref[:, 0:1]
    w = idx_ref[:, 1:2]
    t = idx_ref[:, 2:3]
    ohm = (m == lax.broadcasted_iota(jnp.int32, (_BN, 8), 1)).astype(f32)
    ohw = (w == lax.broadcasted_iota(jnp.int32, (_BN, 7), 1)).astype(f32)
    oht = (t == lax.broadcasted_iota(jnp.int32, (_BN, 512), 1)).astype(f32)
    tm = jnp.dot(me_ref[...], w1m_ref[...], preferred_element_type=f32)
    tw = jnp.dot(we_ref[...], w1w_ref[...], preferred_element_type=f32)
    tt = jnp.dot(te_ref[...], w1t_ref[...], preferred_element_type=f32)
    z1 = (jnp.dot(ohm, tm, preferred_element_type=f32)
          + jnp.dot(ohw, tw, preferred_element_type=f32)
          + jnp.dot(oht, tt, preferred_element_type=f32)
          + jnp.dot(cont_ref[...], w1c_ref[...], preferred_element_type=f32)
          + b1_ref[...])
    h1 = jnp.maximum(z1, 0.0)
    z2 = jnp.dot(h1, w2t_ref[...], preferred_element_type=f32) + b2_ref[...]
    out_ref[...] = jnp.maximum(z2, 0.0)


def _input_stage(idx3, cont, me, we, te, w1m, w1w, w1t, w1c, b1, w2t, b2):
    full = lambda shape: pl.BlockSpec(shape, lambda i: (0, 0))
    return pl.pallas_call(
        _input_body,
        grid=(_NB,),
        in_specs=[
            pl.BlockSpec((_BN, 3), lambda i: (i, 0)),
            pl.BlockSpec((_BN, _CONT), lambda i: (i, 0)),
            full((8, _EMB)), full((7, _EMB)), full((512, _EMB)),
            full((_EMB, _D)), full((_EMB, _D)), full((_EMB, _D)),
            full((_CONT, _D)), full((1, _D)), full((_D, _D)), full((1, _D)),
        ],
        out_specs=pl.BlockSpec((_BN, _D), lambda i: (i, 0)),
        out_shape=jax.ShapeDtypeStruct((_N, _D), jnp.float32),
    )(idx3, cont, me, we, te, w1m, w1w, w1t, w1c, b1, w2t, b2)


def _segsum_body(h_hbm, col_hbm, row_hbm, val_hbm, zero_hbm, dumm_hbm, out_hbm,
                 col_sc, row_sc, valv, buf0, buf1, acc,
                 gsem0, gsem1, ssem0, ssem1):
    cid = lax.axis_index("c")
    sid = lax.axis_index("s")
    wid = cid * _TILES + sid

    pltpu.sync_copy(zero_hbm, acc.at[pl.ds(sid * _RPT, _RPT)])
    plsc.subcore_barrier()

    _T = _SCH // 2

    def drain(buf, sem):
        pltpu.make_async_copy(dumm_hbm, buf, sem).wait()

    def scale(buf, j):
        def emit_edge(vg, e, k):
            wv = jnp.full((16,), vg[k], jnp.float32)
            for c in range(_D // 16):
                sl = pl.ds(c * 16, 16)
                buf[e, sl] = buf[e, sl] * wv

        def group_body(g, carry):
            vg = valv[pl.ds(j * 128 + g * 16, 16)]
            for k in range(16):
                emit_edge(vg, g * 16 + k, k)
            return carry
        lax.fori_loop(0, 7, group_body, 0)
        vg = valv[pl.ds(j * 128 + 112, 16)]
        for k in range(_CHUNK - 112):
            emit_edge(vg, 112 + k, k)

    def sch_body(s, carry):
        cbase = wid * _NCH + s * _SCH
        pltpu.sync_copy(col_hbm.at[pl.ds(cbase, _SCH)], col_sc)
        pltpu.sync_copy(row_hbm.at[pl.ds(cbase, _SCH)], row_sc)
        pltpu.sync_copy(val_hbm.at[pl.ds(cbase * 128, _SCH * 128)], valv)
        pltpu.async_copy(h_hbm.at[col_sc.at[0]],
                         buf0.at[pl.ds(0, _CHUNK)], gsem0)

        def pair_body(t, carry2):
            j0 = 2 * t
            j1 = 2 * t + 1

            @pl.when(t > 0)
            def _():
                drain(buf1, ssem1)
            pltpu.async_copy(h_hbm.at[col_sc.at[j0 + 1]],
                             buf1.at[pl.ds(0, _CHUNK)], gsem1)
            drain(buf0, gsem0)
            scale(buf0, j0)
            pltpu.async_copy(buf0.at[pl.ds(0, _CHUNK)],
                             acc.at[row_sc.at[j0]], ssem0, add=True)

            @pl.when(t < _T - 1)
            def _():
                drain(buf0, ssem0)
                pltpu.async_copy(h_hbm.at[col_sc.at[j1 + 1]],
                                 buf0.at[pl.ds(0, _CHUNK)], gsem0)
            drain(buf1, gsem1)
            scale(buf1, j1)
            pltpu.async_copy(buf1.at[pl.ds(0, _CHUNK)],
                             acc.at[row_sc.at[j1]], ssem1, add=True)
            return carry2
        lax.fori_loop(0, _T, pair_body, 0)
        drain(buf1, ssem1)
        drain(buf0, ssem0)
        return carry
    lax.fori_loop(0, _NSCH, sch_body, 0)

    plsc.subcore_barrier()
    pltpu.sync_copy(acc.at[pl.ds(sid * _RPT, _RPT)],
                    out_hbm.at[cid, pl.ds(sid * _RPT, _RPT)])


def _make_segsum():
    mesh = plsc.VectorSubcoreMesh(core_axis_name="c", subcore_axis_name="s")
    return functools.partial(
        pl.kernel, _segsum_body, mesh=mesh,
        out_type=jax.ShapeDtypeStruct((2, _NPAD, _D), jnp.float32),
        scratch_types=[
            pltpu.VMEM((_SCH, _CHUNK), jnp.int32),
            pltpu.VMEM((_SCH, _CHUNK), jnp.int32),
            pltpu.VMEM((_SCH * 128,), jnp.float32),
            pltpu.VMEM((_CHUNK, _D), jnp.float32),
            pltpu.VMEM((_CHUNK, _D), jnp.float32),
            pltpu.VMEM_SHARED((_NPAD, _D), jnp.float32),
            pltpu.SemaphoreType.DMA,
            pltpu.SemaphoreType.DMA,
            pltpu.SemaphoreType.DMA,
            pltpu.SemaphoreType.DMA,
        ],
    )()


def _layer_body(h_ref, p_ref, swt_ref, sb_ref, nwt_ref, nb_ref, out_ref):
    f32 = jnp.float32
    h = h_ref[...]
    neigh = p_ref[0] + p_ref[1]
    z = (jnp.dot(h, swt_ref[...], preferred_element_type=f32) + sb_ref[...]
         + jnp.dot(neigh, nwt_ref[...], preferred_element_type=f32)
         + nb_ref[...])
    out_ref[...] = h + jnp.maximum(z, 0.0)


def _layer_final_body(h_ref, p_ref, swt_ref, sb_ref, nwt_ref, nb_ref,
                      out_ref, gsum_ref, gmax_ref):
    f32 = jnp.float32
    i = pl.program_id(0)
    h = h_ref[...]
    neigh = p_ref[0] + p_ref[1]
    z = (jnp.dot(h, swt_ref[...], preferred_element_type=f32) + sb_ref[...]
         + jnp.dot(neigh, nwt_ref[...], preferred_element_type=f32)
         + nb_ref[...])
    hn = h + jnp.maximum(z, 0.0)
    out_ref[...] = hn
    bs = jnp.sum(hn, axis=0, keepdims=True)
    bm = jnp.max(hn, axis=0, keepdims=True)

    @pl.when(i == 0)
    def _():
        gsum_ref[...] = bs
        gmax_ref[...] = bm

    @pl.when(i > 0)
    def _():
        gsum_ref[...] = gsum_ref[...] + bs
        gmax_ref[...] = jnp.maximum(gmax_ref[...], bm)

    @pl.when(i == _NB - 1)
    def _():
        gsum_ref[...] = gsum_ref[...] * (1.0 / _N)


def _layer(h, p, swt, sb, nwt, nb, final):
    full = lambda shape: pl.BlockSpec(shape, lambda i: (0, 0))
    in_specs = [
        pl.BlockSpec((_BN, _D), lambda i: (i, 0)),
        pl.BlockSpec((2, _BN, _D), lambda i: (0, i, 0)),
        full((_D, _D)), full((1, _D)), full((_D, _D)), full((1, _D)),
    ]
    if not final:
        return pl.pallas_call(
            _layer_body, grid=(_NB,), in_specs=in_specs,
            out_specs=pl.BlockSpec((_BN, _D), lambda i: (i, 0)),
            out_shape=jax.ShapeDtypeStruct((_N, _D), jnp.float32),
        )(h, p, swt, sb, nwt, nb)
    return pl.pallas_call(
        _layer_final_body, grid=(_NB,), in_specs=in_specs,
        out_specs=[
            pl.BlockSpec((_BN, _D), lambda i: (i, 0)),
            pl.BlockSpec((1, _D), lambda i: (0, 0)),
            pl.BlockSpec((1, _D), lambda i: (0, 0)),
        ],
        out_shape=[
            jax.ShapeDtypeStruct((_N, _D), jnp.float32),
            jax.ShapeDtypeStruct((1, _D), jnp.float32),
            jax.ShapeDtypeStruct((1, _D), jnp.float32),
        ],
    )(h, p, swt, sb, nwt, nb)


_segsum = None


def kernel(machine_idx, weekday_idx, type_idx, cont_feat, a_indices, a_values,
           machine_emb, weekday_emb, type_emb, W_in1, b_in1, W_in2, b_in2,
           self_W, self_b, neigh_W, neigh_b):
    global _segsum
    if _segsum is None:
        _segsum = _make_segsum()

    row = a_indices[0]
    col = a_indices[1]
    idx3 = jnp.stack([machine_idx.astype(jnp.int32),
                      weekday_idx.astype(jnp.int32),
                      type_idx.astype(jnp.int32)], axis=1)

    w1m = W_in1[:, 0:_EMB].T
    w1w = W_in1[:, _EMB:2 * _EMB].T
    w1t = W_in1[:, 2 * _EMB:3 * _EMB].T
    w1c = W_in1[:, 3 * _EMB:].T
    b1 = b_in1.reshape(1, _D)
    w2t = W_in2.T
    b2 = b_in2.reshape(1, _D)

    h = _input_stage(idx3, cont_feat, machine_emb, weekday_emb,
                     type_emb, w1m, w1w, w1t, w1c, b1, w2t, b2)

    zeros = jnp.zeros((_RPT, _D), jnp.float32)
    dummy = jnp.zeros((_CHUNK, _D), jnp.float32)
    col2 = col.reshape(_NCHT, _CHUNK)
    row2 = row.reshape(_NCHT, _CHUNK)
    vals2 = jnp.pad(a_values.reshape(_NCHT, _CHUNK),
                    ((0, 0), (0, 128 - _CHUNK))).reshape(_NCHT * 128)
    for l in range(2):
        p = _segsum(h, col2, row2, vals2, zeros, dummy)
        swt = self_W[l].T
        nwt = neigh_W[l].T
        sb = self_b[l].reshape(1, _D)
        nb = neigh_b[l].reshape(1, _D)
        if l == 0:
            h = _layer(h, p, swt, sb, nwt, nb, final=False)
        else:
            h, gsum, gmax = _layer(h, p, swt, sb, nwt, nb, final=True)

    g = jnp.concatenate([gsum[0], gmax[0]], axis=0)
    return (g, h)


# speedup vs baseline: 1.0360x; 1.0343x over previous
import functools

import jax
import jax.numpy as jnp

from jax import lax
from jax.experimental import pallas as pl
from jax.experimental.pallas import tpu as pltpu
from jax.experimental.pallas import tpu_sc as plsc

_N = 10000
_E = 320000
_D = 128
_EMB = 16
_CONT = 12

_BN = 2000
_NB = _N // _BN

_WORKERS = 32
_CHUNK = 125
_NCHT = _E // _CHUNK
_NCH = _NCHT // _WORKERS
_SCH = 8
_NSCH = _NCH // _SCH
_TILES = 16
_NPAD = 10240
_RPT = _NPAD // _TILES


def _input_body(idx_ref, cont_ref, me_ref, we_ref, te_ref,
                w1m_ref, w1w_ref, w1t_ref, w1c_ref, b1_ref, w2t_ref, b2_ref,
                out_ref):
    f32 = jnp.float32
    m = idx_ref[:, 0:1]
    w = idx_ref[:, 1:2]
    t = idx_ref[:, 2:3]
    ohm = (m == lax.broadcasted_iota(jnp.int32, (_BN, 8), 1)).astype(f32)
    ohw = (w == lax.broadcasted_iota(jnp.int32, (_BN, 7), 1)).astype(f32)
    oht = (t == lax.broadcasted_iota(jnp.int32, (_BN, 512), 1)).astype(f32)
    tm = jnp.dot(me_ref[...], w1m_ref[...], preferred_element_type=f32)
    tw = jnp.dot(we_ref[...], w1w_ref[...], preferred_element_type=f32)
    tt = jnp.dot(te_ref[...], w1t_ref[...], preferred_element_type=f32)
    z1 = (jnp.dot(ohm, tm, preferred_element_type=f32)
          + jnp.dot(ohw, tw, preferred_element_type=f32)
          + jnp.dot(oht, tt, preferred_element_type=f32)
          + jnp.dot(cont_ref[...], w1c_ref[...], preferred_element_type=f32)
          + b1_ref[...])
    h1 = jnp.maximum(z1, 0.0)
    z2 = jnp.dot(h1, w2t_ref[...], preferred_element_type=f32) + b2_ref[...]
    out_ref[...] = jnp.maximum(z2, 0.0)


def _input_stage(idx3, cont, me, we, te, w1m, w1w, w1t, w1c, b1, w2t, b2):
    full = lambda shape: pl.BlockSpec(shape, lambda i: (0, 0))
    return pl.pallas_call(
        _input_body,
        grid=(_NB,),
        in_specs=[
            pl.BlockSpec((_BN, 3), lambda i: (i, 0)),
            pl.BlockSpec((_BN, _CONT), lambda i: (i, 0)),
            full((8, _EMB)), full((7, _EMB)), full((512, _EMB)),
            full((_EMB, _D)), full((_EMB, _D)), full((_EMB, _D)),
            full((_CONT, _D)), full((1, _D)), full((_D, _D)), full((1, _D)),
        ],
        out_specs=pl.BlockSpec((_BN, _D), lambda i: (i, 0)),
        out_shape=jax.ShapeDtypeStruct((_N, _D), jnp.float32),
    )(idx3, cont, me, we, te, w1m, w1w, w1t, w1c, b1, w2t, b2)


def _segsum_body(h_hbm, a3_hbm, val_hbm, zero_hbm, dumm_hbm, out_hbm,
                 col_sc0, row_sc0, valv0, col_sc1, row_sc1, valv1,
                 buf0, buf1, acc,
                 gsem0, gsem1, ssem0, ssem1, stsem0, stsem1):
    cid = lax.axis_index("c")
    sid = lax.axis_index("s")
    wid = cid * _TILES + sid

    pltpu.sync_copy(zero_hbm, acc.at[pl.ds(sid * _RPT, _RPT)])
    plsc.subcore_barrier()

    _T = _SCH // 2
    sts = ((col_sc0, row_sc0, valv0, stsem0),
           (col_sc1, row_sc1, valv1, stsem1))

    def drain(buf, sem):
        pltpu.make_async_copy(dumm_hbm, buf, sem).wait()

    def stage(s, st):
        col_sc, row_sc, valv, stsem = st
        cbase = wid * _NCH + s * _SCH
        pltpu.async_copy(a3_hbm.at[1, pl.ds(cbase, _SCH)], col_sc, stsem)
        pltpu.async_copy(a3_hbm.at[0, pl.ds(cbase, _SCH)], row_sc, stsem)
        pltpu.async_copy(val_hbm.at[pl.ds(cbase * 128, _SCH * 128)], valv,
                         stsem)

    def stage_drain(st):
        col_sc, row_sc, valv, stsem = st
        pltpu.make_async_copy(a3_hbm.at[1, pl.ds(0, _SCH)], col_sc,
                              stsem).wait()
        pltpu.make_async_copy(a3_hbm.at[0, pl.ds(0, _SCH)], row_sc,
                              stsem).wait()
        pltpu.make_async_copy(val_hbm.at[pl.ds(0, _SCH * 128)], valv,
                              stsem).wait()

    def scale(buf, valv, j):
        def emit_edge(vg, e, k):
            wv = jnp.full((16,), vg[k], jnp.float32)
            for c in range(_D // 16):
                sl = pl.ds(c * 16, 16)
                buf[e, sl] = buf[e, sl] * wv

        def group_body(g, carry):
            vg = valv[pl.ds(j * 128 + g * 16, 16)]
            for k in range(16):
                emit_edge(vg, g * 16 + k, k)
            return carry
        lax.fori_loop(0, 7, group_body, 0)
        vg = valv[pl.ds(j * 128 + 112, 16)]
        for k in range(_CHUNK - 112):
            emit_edge(vg, 112 + k, k)

    def sch_half(s, st, stn):
        col_sc, row_sc, valv, stsem = st
        stage_drain(st)
        pltpu.async_copy(h_hbm.at[col_sc.at[0]],
                         buf0.at[pl.ds(0, _CHUNK)], gsem0)

        @pl.when(s + 1 < _NSCH)
        def _():
            stage(s + 1, stn)

        def pair_body(t, carry2):
            j0 = 2 * t
            j1 = 2 * t + 1

            @pl.when(t > 0)
            def _():
                drain(buf1, ssem1)
            pltpu.async_copy(h_hbm.at[col_sc.at[j0 + 1]],
                             buf1.at[pl.ds(0, _CHUNK)], gsem1)
            drain(buf0, gsem0)
            scale(buf0, valv, j0)
            pltpu.async_copy(buf0.at[pl.ds(0, _CHUNK)],
                             acc.at[row_sc.at[j0]], ssem0, add=True)

            @pl.when(t < _T - 1)
            def _():
                drain(buf0, ssem0)
                pltpu.async_copy(h_hbm.at[col_sc.at[j1 + 1]],
                                 buf0.at[pl.ds(0, _CHUNK)], gsem0)
            drain(buf1, gsem1)
            scale(buf1, valv, j1)
            pltpu.async_copy(buf1.at[pl.ds(0, _CHUNK)],
                             acc.at[row_sc.at[j1]], ssem1, add=True)
            return carry2
        lax.fori_loop(0, _T, pair_body, 0)
        drain(buf1, ssem1)
        drain(buf0, ssem0)

    stage(0, sts[0])

    def sch_pair(u, carry):
        sch_half(2 * u, sts[0], sts[1])
        sch_half(2 * u + 1, sts[1], sts[0])
        return carry
    lax.fori_loop(0, _NSCH // 2, sch_pair, 0)

    plsc.subcore_barrier()
    pltpu.sync_copy(acc.at[pl.ds(sid * _RPT, _RPT)],
                    out_hbm.at[cid, pl.ds(sid * _RPT, _RPT)])


def _make_segsum():
    mesh = plsc.VectorSubcoreMesh(core_axis_name="c", subcore_axis_name="s")
    return functools.partial(
        pl.kernel, _segsum_body, mesh=mesh,
        out_type=jax.ShapeDtypeStruct((2, _NPAD, _D), jnp.float32),
        scratch_types=[
            pltpu.VMEM((_SCH, _CHUNK), jnp.int32),
            pltpu.VMEM((_SCH, _CHUNK), jnp.int32),
            pltpu.VMEM((_SCH * 128,), jnp.float32),
            pltpu.VMEM((_SCH, _CHUNK), jnp.int32),
            pltpu.VMEM((_SCH, _CHUNK), jnp.int32),
            pltpu.VMEM((_SCH * 128,), jnp.float32),
            pltpu.VMEM((_CHUNK, _D), jnp.float32),
            pltpu.VMEM((_CHUNK, _D), jnp.float32),
            pltpu.VMEM_SHARED((_NPAD, _D), jnp.float32),
            pltpu.SemaphoreType.DMA,
            pltpu.SemaphoreType.DMA,
            pltpu.SemaphoreType.DMA,
            pltpu.SemaphoreType.DMA,
            pltpu.SemaphoreType.DMA,
            pltpu.SemaphoreType.DMA,
        ],
    )()


def _layer_body(h_ref, p_ref, swt_ref, sb_ref, nwt_ref, nb_ref, out_ref):
    f32 = jnp.float32
    h = h_ref[...]
    neigh = p_ref[0] + p_ref[1]
    z = (jnp.dot(h, swt_ref[...], preferred_element_type=f32) + sb_ref[...]
         + jnp.dot(neigh, nwt_ref[...], preferred_element_type=f32)
         + nb_ref[...])
    out_ref[...] = h + jnp.maximum(z, 0.0)


def _layer_final_body(h_ref, p_ref, swt_ref, sb_ref, nwt_ref, nb_ref,
                      out_ref, gsum_ref, gmax_ref):
    f32 = jnp.float32
    i = pl.program_id(0)
    h = h_ref[...]
    neigh = p_ref[0] + p_ref[1]
    z = (jnp.dot(h, swt_ref[...], preferred_element_type=f32) + sb_ref[...]
         + jnp.dot(neigh, nwt_ref[...], preferred_element_type=f32)
         + nb_ref[...])
    hn = h + jnp.maximum(z, 0.0)
    out_ref[...] = hn
    bs = jnp.sum(hn, axis=0, keepdims=True)
    bm = jnp.max(hn, axis=0, keepdims=True)

    @pl.when(i == 0)
    def _():
        gsum_ref[...] = bs
        gmax_ref[...] = bm

    @pl.when(i > 0)
    def _():
        gsum_ref[...] = gsum_ref[...] + bs
        gmax_ref[...] = jnp.maximum(gmax_ref[...], bm)

    @pl.when(i == _NB - 1)
    def _():
        gsum_ref[...] = gsum_ref[...] * (1.0 / _N)


def _layer(h, p, swt, sb, nwt, nb, final):
    full = lambda shape: pl.BlockSpec(shape, lambda i: (0, 0))
    in_specs = [
        pl.BlockSpec((_BN, _D), lambda i: (i, 0)),
        pl.BlockSpec((2, _BN, _D), lambda i: (0, i, 0)),
        full((_D, _D)), full((1, _D)), full((_D, _D)), full((1, _D)),
    ]
    if not final:
        return pl.pallas_call(
            _layer_body, grid=(_NB,), in_specs=in_specs,
            out_specs=pl.BlockSpec((_BN, _D), lambda i: (i, 0)),
            out_shape=jax.ShapeDtypeStruct((_N, _D), jnp.float32),
        )(h, p, swt, sb, nwt, nb)
    return pl.pallas_call(
        _layer_final_body, grid=(_NB,), in_specs=in_specs,
        out_specs=[
            pl.BlockSpec((_BN, _D), lambda i: (i, 0)),
            pl.BlockSpec((1, _D), lambda i: (0, 0)),
            pl.BlockSpec((1, _D), lambda i: (0, 0)),
        ],
        out_shape=[
            jax.ShapeDtypeStruct((_N, _D), jnp.float32),
            jax.ShapeDtypeStruct((1, _D), jnp.float32),
            jax.ShapeDtypeStruct((1, _D), jnp.float32),
        ],
    )(h, p, swt, sb, nwt, nb)


_segsum = None


def kernel(machine_idx, weekday_idx, type_idx, cont_feat, a_indices, a_values,
           machine_emb, weekday_emb, type_emb, W_in1, b_in1, W_in2, b_in2,
           self_W, self_b, neigh_W, neigh_b):
    global _segsum
    if _segsum is None:
        _segsum = _make_segsum()

    idx3 = jnp.stack([machine_idx.astype(jnp.int32),
                      weekday_idx.astype(jnp.int32),
                      type_idx.astype(jnp.int32)], axis=1)

    w1m = W_in1[:, 0:_EMB].T
    w1w = W_in1[:, _EMB:2 * _EMB].T
    w1t = W_in1[:, 2 * _EMB:3 * _EMB].T
    w1c = W_in1[:, 3 * _EMB:].T
    b1 = b_in1.reshape(1, _D)
    w2t = W_in2.T
    b2 = b_in2.reshape(1, _D)

    h = _input_stage(idx3, cont_feat, machine_emb, weekday_emb,
                     type_emb, w1m, w1w, w1t, w1c, b1, w2t, b2)

    zeros = jnp.zeros((_RPT, _D), jnp.float32)
    dummy = jnp.zeros((_CHUNK, _D), jnp.float32)
    a3 = a_indices.reshape(2, _NCHT, _CHUNK)
    vals2 = jnp.pad(a_values.reshape(_NCHT, _CHUNK),
                    ((0, 0), (0, 128 - _CHUNK))).reshape(_NCHT * 128)
    for l in range(2):
        p = _segsum(h, a3, vals2, zeros, dummy)
        swt = self_W[l].T
        nwt = neigh_W[l].T
        sb = self_b[l].reshape(1, _D)
        nb = neigh_b[l].reshape(1, _D)
        if l == 0:
            h = _layer(h, p, swt, sb, nwt, nb, final=False)
        else:
            h, gsum, gmax = _layer(h, p, swt, sb, nwt, nb, final=True)

    g = jnp.concatenate([gsum[0], gmax[0]], axis=0)
    return (g, h)
